# baseline (speedup 1.0000x reference)
import functools

import jax
import jax.numpy as jnp
from jax import lax
from jax.experimental import pallas as pl
from jax.experimental.pallas import tpu as pltpu
from jax.experimental.pallas import tpu_sc as plsc

L = 16
NC = 2
NS = 16
NW = NC * NS

ROWS, COLS = 4096, 8192
RPW = ROWS // NW
BR, BC = 8, 2048
NSLAB = RPW // BR
NCOL = COLS // BC
NCHUNK = NSLAB * NCOL
NPAIR = NCHUNK // 2
NK = 1024
UNROLL = 2


def _body(x_hbm, coeffs_hbm, k0_hbm, invh_hbm, out_hbm,
          coeffs_v, a_v, b_v, k0_v, invh_v, xbuf_a, xbuf_b, obuf_a, obuf_b,
          insem_a, insem_b, outsem_a, outsem_b):
    wid = lax.axis_index("s") * NC + lax.axis_index("c")
    row_base = wid * RPW

    pltpu.sync_copy(coeffs_hbm, coeffs_v)
    pltpu.sync_copy(k0_hbm, k0_v)
    pltpu.sync_copy(invh_hbm, invh_v)

    k0 = k0_v[...]
    invh = invh_v[...]
    k0invh = k0 * invh

    @plsc.parallel_loop(0, NK, step=L, unroll=2)
    def _tab(j):
        iv = lax.iota(jnp.int32, L) + j
        c0 = plsc.load_gather(coeffs_v, [iv])
        c1 = plsc.load_gather(coeffs_v, [jnp.minimum(iv + 1, NK - 1)])
        s = c1 - c0
        a_v[pl.ds(j, L)] = c0 - (k0invh + iv.astype(jnp.float32)) * s
        b_v[pl.ds(j, L)] = s * invh

    def compute(xbuf, obuf):
        @plsc.parallel_loop(0, BC, step=L, unroll=UNROLL)
        def _vec(j):
            for r in range(BR):
                xv = xbuf[r, pl.ds(j, L)]
                g = (xv - k0) * invh
                i = jnp.clip(g.astype(jnp.int32), 0, NK - 2)
                av = plsc.load_gather(a_v, [i])
                bv = plsc.load_gather(b_v, [i])
                obuf[r, pl.ds(j, L)] = av + bv * xv

    def in_slice(ci):
        r0 = row_base + (ci // NCOL) * BR
        c0 = (ci % NCOL) * BC
        return x_hbm.at[pl.ds(r0, BR), pl.ds(c0, BC)]

    def out_slice(ci):
        r0 = row_base + (ci // NCOL) * BR
        c0 = (ci % NCOL) * BC
        return out_hbm.at[pl.ds(r0, BR), pl.ds(c0, BC)]

    pltpu.async_copy(in_slice(0), xbuf_a, insem_a)

    def pair(k, carry):
        ci0 = 2 * k
        ci1 = 2 * k + 1

        pltpu.async_copy(in_slice(ci1), xbuf_b, insem_b)
        pltpu.make_async_copy(in_slice(ci0), xbuf_a, insem_a).wait()

        @pl.when(k > 0)
        def _():
            pltpu.make_async_copy(obuf_a, out_slice(ci0), outsem_a).wait()

        compute(xbuf_a, obuf_a)
        pltpu.async_copy(obuf_a, out_slice(ci0), outsem_a)

        @pl.when(k + 1 < NPAIR)
        def _():
            pltpu.async_copy(in_slice(ci0 + 2), xbuf_a, insem_a)

        pltpu.make_async_copy(in_slice(ci1), xbuf_b, insem_b).wait()

        @pl.when(k > 0)
        def _():
            pltpu.make_async_copy(obuf_b, out_slice(ci1), outsem_b).wait()

        compute(xbuf_b, obuf_b)
        pltpu.async_copy(obuf_b, out_slice(ci1), outsem_b)
        return carry

    lax.fori_loop(0, NPAIR, pair, 0)

    pltpu.make_async_copy(obuf_a, out_slice(NCHUNK - 2), outsem_a).wait()
    pltpu.make_async_copy(obuf_b, out_slice(NCHUNK - 1), outsem_b).wait()


@functools.partial(
    pl.kernel,
    out_type=jax.ShapeDtypeStruct((ROWS, COLS), jnp.float32),
    mesh=plsc.VectorSubcoreMesh(
        core_axis_name="c", subcore_axis_name="s",
        num_cores=NC, num_subcores=NS),
    compiler_params=pltpu.CompilerParams(needs_layout_passes=False),
    scratch_types=[
        pltpu.VMEM((NK,), jnp.float32),
        pltpu.VMEM((NK,), jnp.float32),
        pltpu.VMEM((NK,), jnp.float32),
        pltpu.VMEM((L,), jnp.float32),
        pltpu.VMEM((L,), jnp.float32),
        pltpu.VMEM((BR, BC), jnp.float32),
        pltpu.VMEM((BR, BC), jnp.float32),
        pltpu.VMEM((BR, BC), jnp.float32),
        pltpu.VMEM((BR, BC), jnp.float32),
        pltpu.SemaphoreType.DMA,
        pltpu.SemaphoreType.DMA,
        pltpu.SemaphoreType.DMA,
        pltpu.SemaphoreType.DMA,
    ],
)
def _spline_sc(*refs):
    _body(*refs)


def kernel(x, coeffs, knots):
    k0 = knots[0]
    invh = (NK - 1) / (knots[-1] - k0)
    k0s = jnp.full((L,), k0, jnp.float32)
    invhs = jnp.full((L,), invh, jnp.float32)
    return _spline_sc(x, coeffs, k0s, invhs)


# speedup vs baseline: 1.0560x; 1.0560x over previous
import functools

import jax
import jax.numpy as jnp
from jax import lax
from jax.experimental import pallas as pl
from jax.experimental.pallas import tpu as pltpu
from jax.experimental.pallas import tpu_sc as plsc

L = 16
NC = 2
NS = 16
NW = NC * NS

ROWS, COLS = 4096, 8192
RPW = ROWS // NW
BR, BC = 8, 2048
NSLAB = RPW // BR
NCOL = COLS // BC
NCHUNK = NSLAB * NCOL
NPAIR = NCHUNK // 2
NK = 1024
UNROLL = 1


def _body(x_hbm, coeffs_hbm, k0_hbm, invh_hbm, out_hbm,
          coeffs_v, a_v, b_v, k0_v, invh_v, xbuf_a, xbuf_b, obuf_a, obuf_b,
          insem_a, insem_b, outsem_a, outsem_b):
    wid = lax.axis_index("s") * NC + lax.axis_index("c")
    row_base = wid * RPW

    pltpu.async_copy(x_hbm.at[pl.ds(row_base, BR), pl.ds(0, BC)], xbuf_a,
                     insem_a)
    pltpu.async_copy(coeffs_hbm, coeffs_v, outsem_a)
    pltpu.async_copy(k0_hbm, k0_v, outsem_b)
    pltpu.async_copy(invh_hbm, invh_v, insem_b)
    pltpu.make_async_copy(coeffs_hbm, coeffs_v, outsem_a).wait()
    pltpu.make_async_copy(k0_hbm, k0_v, outsem_b).wait()
    pltpu.make_async_copy(invh_hbm, invh_v, insem_b).wait()

    k0 = k0_v[...]
    invh = invh_v[...]
    k0invh = k0 * invh

    @plsc.parallel_loop(0, NK, step=L, unroll=2)
    def _tab(j):
        iv = lax.iota(jnp.int32, L) + j
        c0 = plsc.load_gather(coeffs_v, [iv])
        c1 = plsc.load_gather(coeffs_v, [jnp.minimum(iv + 1, NK - 1)])
        s = c1 - c0
        a_v[pl.ds(j, L)] = c0 - (k0invh + iv.astype(jnp.float32)) * s
        b_v[pl.ds(j, L)] = s * invh

    def compute(xbuf, obuf):
        @plsc.parallel_loop(0, BC, step=L, unroll=UNROLL)
        def _vec(j):
            for r in range(BR):
                xv = xbuf[r, pl.ds(j, L)]
                g = (xv - k0) * invh
                i = jnp.clip(g.astype(jnp.int32), 0, NK - 2)
                av = plsc.load_gather(a_v, [i])
                bv = plsc.load_gather(b_v, [i])
                obuf[r, pl.ds(j, L)] = av + bv * xv

    def in_slice(ci):
        r0 = row_base + (ci // NCOL) * BR
        c0 = (ci % NCOL) * BC
        return x_hbm.at[pl.ds(r0, BR), pl.ds(c0, BC)]

    def out_slice(ci):
        r0 = row_base + (ci // NCOL) * BR
        c0 = (ci % NCOL) * BC
        return out_hbm.at[pl.ds(r0, BR), pl.ds(c0, BC)]

    def pair(k, carry):
        ci0 = 2 * k
        ci1 = 2 * k + 1

        pltpu.async_copy(in_slice(ci1), xbuf_b, insem_b)
        pltpu.make_async_copy(in_slice(ci0), xbuf_a, insem_a).wait()

        @pl.when(k > 0)
        def _():
            pltpu.make_async_copy(obuf_a, out_slice(ci0), outsem_a).wait()

        compute(xbuf_a, obuf_a)
        pltpu.async_copy(obuf_a, out_slice(ci0), outsem_a)

        @pl.when(k + 1 < NPAIR)
        def _():
            pltpu.async_copy(in_slice(ci0 + 2), xbuf_a, insem_a)

        pltpu.make_async_copy(in_slice(ci1), xbuf_b, insem_b).wait()

        @pl.when(k > 0)
        def _():
            pltpu.make_async_copy(obuf_b, out_slice(ci1), outsem_b).wait()

        compute(xbuf_b, obuf_b)
        pltpu.async_copy(obuf_b, out_slice(ci1), outsem_b)
        return carry

    lax.fori_loop(0, NPAIR, pair, 0)

    pltpu.make_async_copy(obuf_a, out_slice(NCHUNK - 2), outsem_a).wait()
    pltpu.make_async_copy(obuf_b, out_slice(NCHUNK - 1), outsem_b).wait()


@functools.partial(
    pl.kernel,
    out_type=jax.ShapeDtypeStruct((ROWS, COLS), jnp.float32),
    mesh=plsc.VectorSubcoreMesh(
        core_axis_name="c", subcore_axis_name="s",
        num_cores=NC, num_subcores=NS),
    compiler_params=pltpu.CompilerParams(needs_layout_passes=False),
    scratch_types=[
        pltpu.VMEM((NK,), jnp.float32),
        pltpu.VMEM((NK,), jnp.float32),
        pltpu.VMEM((NK,), jnp.float32),
        pltpu.VMEM((L,), jnp.float32),
        pltpu.VMEM((L,), jnp.float32),
        pltpu.VMEM((BR, BC), jnp.float32),
        pltpu.VMEM((BR, BC), jnp.float32),
        pltpu.VMEM((BR, BC), jnp.float32),
        pltpu.VMEM((BR, BC), jnp.float32),
        pltpu.SemaphoreType.DMA,
        pltpu.SemaphoreType.DMA,
        pltpu.SemaphoreType.DMA,
        pltpu.SemaphoreType.DMA,
    ],
)
def _spline_sc(*refs):
    _body(*refs)


def kernel(x, coeffs, knots):
    k0 = knots[0]
    invh = (NK - 1) / (knots[-1] - k0)
    k0s = jnp.full((L,), k0, jnp.float32)
    invhs = jnp.full((L,), invh, jnp.float32)
    return _spline_sc(x, coeffs, k0s, invhs)


# speedup vs baseline: 1.1023x; 1.0438x over previous
import functools

import jax
import jax.numpy as jnp
from jax import lax
from jax.experimental import pallas as pl
from jax.experimental.pallas import tpu as pltpu
from jax.experimental.pallas import tpu_sc as plsc

L = 16
NC = 2
NS = 16
NW = NC * NS

ROWS, COLS = 4096, 8192
RPW = ROWS // NW
BR, BC = 2, 8192
NSLAB = RPW // BR
NCOL = COLS // BC
NCHUNK = NSLAB * NCOL
NPAIR = NCHUNK // 2
NK = 1024
UNROLL = 4


def _body(x_hbm, coeffs_hbm, k0_hbm, invh_hbm, out_hbm,
          coeffs_v, a_v, b_v, k0_v, invh_v, xbuf_a, xbuf_b, obuf_a, obuf_b,
          insem_a, insem_b, outsem_a, outsem_b):
    wid = lax.axis_index("s") * NC + lax.axis_index("c")
    row_base = wid * RPW

    pltpu.async_copy(x_hbm.at[pl.ds(row_base, BR), pl.ds(0, BC)], xbuf_a,
                     insem_a)
    pltpu.async_copy(coeffs_hbm, coeffs_v, outsem_a)
    pltpu.async_copy(k0_hbm, k0_v, outsem_b)
    pltpu.async_copy(invh_hbm, invh_v, insem_b)
    pltpu.make_async_copy(coeffs_hbm, coeffs_v, outsem_a).wait()
    pltpu.make_async_copy(k0_hbm, k0_v, outsem_b).wait()
    pltpu.make_async_copy(invh_hbm, invh_v, insem_b).wait()

    k0 = k0_v[...]
    invh = invh_v[...]
    k0invh = k0 * invh

    @plsc.parallel_loop(0, NK, step=L, unroll=2)
    def _tab(j):
        iv = lax.iota(jnp.int32, L) + j
        c0 = plsc.load_gather(coeffs_v, [iv])
        c1 = plsc.load_gather(coeffs_v, [jnp.minimum(iv + 1, NK - 1)])
        s = c1 - c0
        a_v[pl.ds(j, L)] = c0 - (k0invh + iv.astype(jnp.float32)) * s
        b_v[pl.ds(j, L)] = s * invh

    def compute(xbuf, obuf):
        @plsc.parallel_loop(0, BC, step=L, unroll=UNROLL)
        def _vec(j):
            for r in range(BR):
                xv = xbuf[r, pl.ds(j, L)]
                g = (xv - k0) * invh
                i = jnp.clip(g.astype(jnp.int32), 0, NK - 2)
                av = plsc.load_gather(a_v, [i])
                bv = plsc.load_gather(b_v, [i])
                obuf[r, pl.ds(j, L)] = av + bv * xv

    def in_slice(ci):
        r0 = row_base + (ci // NCOL) * BR
        c0 = (ci % NCOL) * BC
        return x_hbm.at[pl.ds(r0, BR), pl.ds(c0, BC)]

    def out_slice(ci):
        r0 = row_base + (ci // NCOL) * BR
        c0 = (ci % NCOL) * BC
        return out_hbm.at[pl.ds(r0, BR), pl.ds(c0, BC)]

    def pair(k, carry):
        ci0 = 2 * k
        ci1 = 2 * k + 1

        pltpu.async_copy(in_slice(ci1), xbuf_b, insem_b)
        pltpu.make_async_copy(in_slice(ci0), xbuf_a, insem_a).wait()

        @pl.when(k > 0)
        def _():
            pltpu.make_async_copy(obuf_a, out_slice(ci0), outsem_a).wait()

        compute(xbuf_a, obuf_a)
        pltpu.async_copy(obuf_a, out_slice(ci0), outsem_a)

        @pl.when(k + 1 < NPAIR)
        def _():
            pltpu.async_copy(in_slice(ci0 + 2), xbuf_a, insem_a)

        pltpu.make_async_copy(in_slice(ci1), xbuf_b, insem_b).wait()

        @pl.when(k > 0)
        def _():
            pltpu.make_async_copy(obuf_b, out_slice(ci1), outsem_b).wait()

        compute(xbuf_b, obuf_b)
        pltpu.async_copy(obuf_b, out_slice(ci1), outsem_b)
        return carry

    lax.fori_loop(0, NPAIR, pair, 0)

    pltpu.make_async_copy(obuf_a, out_slice(NCHUNK - 2), outsem_a).wait()
    pltpu.make_async_copy(obuf_b, out_slice(NCHUNK - 1), outsem_b).wait()


@functools.partial(
    pl.kernel,
    out_type=jax.ShapeDtypeStruct((ROWS, COLS), jnp.float32),
    mesh=plsc.VectorSubcoreMesh(
        core_axis_name="c", subcore_axis_name="s",
        num_cores=NC, num_subcores=NS),
    compiler_params=pltpu.CompilerParams(needs_layout_passes=False),
    scratch_types=[
        pltpu.VMEM((NK,), jnp.float32),
        pltpu.VMEM((NK,), jnp.float32),
        pltpu.VMEM((NK,), jnp.float32),
        pltpu.VMEM((L,), jnp.float32),
        pltpu.VMEM((L,), jnp.float32),
        pltpu.VMEM((BR, BC), jnp.float32),
        pltpu.VMEM((BR, BC), jnp.float32),
        pltpu.VMEM((BR, BC), jnp.float32),
        pltpu.VMEM((BR, BC), jnp.float32),
        pltpu.SemaphoreType.DMA,
        pltpu.SemaphoreType.DMA,
        pltpu.SemaphoreType.DMA,
        pltpu.SemaphoreType.DMA,
    ],
)
def _spline_sc(*refs):
    _body(*refs)


def kernel(x, coeffs, knots):
    k0 = knots[0]
    invh = (NK - 1) / (knots[-1] - k0)
    k0s = jnp.full((L,), k0, jnp.float32)
    invhs = jnp.full((L,), invh, jnp.float32)
    return _spline_sc(x, coeffs, k0s, invhs)
